# baseline (speedup 1.0000x reference)
import functools
import jax
import jax.numpy as jnp
from jax import lax
from jax.experimental import pallas as pl
from jax.experimental.pallas import tpu as pltpu

N = 10000
E = 320000
D = 128
NPAD = 10240


def _node_stage(h_ref, wfc_ref, wl_ref, wr_ref, z_ref, asrc_ref, adst_ref):
    z = lax.dot_general(h_ref[...], wfc_ref[...], (((1,), (1,)), ((), ())),
                        preferred_element_type=jnp.float32)
    z = jnp.where(z > 0, z, 0.01 * z)
    z_ref[...] = z
    asrc_ref[...] = jnp.dot(z, wl_ref[...], preferred_element_type=jnp.float32)
    adst_ref[...] = jnp.dot(z, wr_ref[...], preferred_element_type=jnp.float32)


@jax.jit
def kernel(h, edge_index, edge_attr, W_fc, W_attn, W_edge, W_m):
    hp = jnp.zeros((NPAD, D), jnp.float32).at[:N].set(h)
    wl = W_attn[0, :D].reshape(D, 1)
    wr = W_attn[0, D:].reshape(D, 1)

    BR = 1024
    z, asrc, adst = pl.pallas_call(
        _node_stage,
        grid=(NPAD // BR,),
        in_specs=[
            pl.BlockSpec((BR, D), lambda i: (i, 0)),
            pl.BlockSpec((D, D), lambda i: (0, 0)),
            pl.BlockSpec((D, 1), lambda i: (0, 0)),
            pl.BlockSpec((D, 1), lambda i: (0, 0)),
        ],
        out_specs=[
            pl.BlockSpec((BR, D), lambda i: (i, 0)),
            pl.BlockSpec((BR, 1), lambda i: (i, 0)),
            pl.BlockSpec((BR, 1), lambda i: (i, 0)),
        ],
        out_shape=[
            jax.ShapeDtypeStruct((NPAD, D), jnp.float32),
            jax.ShapeDtypeStruct((NPAD, 1), jnp.float32),
            jax.ShapeDtypeStruct((NPAD, 1), jnp.float32),
        ],
    )(hp, W_fc, wl, wr)
    z = z[:N]
    asrc = asrc[:N, 0]
    adst = adst[:N, 0]

    src = edge_index[0]
    dst = edge_index[1]
    a = asrc[src] + adst[dst]
    ev = edge_attr[:, 0] * W_edge[0, 0]
    e = a * ev
    e = jnp.where(e > 0, e, 0.01 * e)
    new_e = e * W_m[0, 0]
    m = jax.ops.segment_max(new_e, dst, num_segments=N)
    m = jnp.where(jnp.isfinite(m), m, 0.0)
    exp_e = jnp.exp(new_e - m[dst])
    denom = jax.ops.segment_sum(exp_e, dst, num_segments=N)
    denom_safe = jnp.where(denom > 0, denom, 1.0)
    alpha = exp_e / denom_safe[dst]
    h_out = jax.ops.segment_sum(alpha[:, None] * z[src], dst, num_segments=N)
    return h_out


# speedup vs baseline: 13.8958x; 13.8958x over previous
import functools
import jax
import jax.numpy as jnp
from jax import lax
from jax.experimental import pallas as pl
from jax.experimental.pallas import tpu as pltpu
from jax.experimental.pallas import tpu_sc as plsc

N = 10000
E = 320000
D = 128
NPAD = 10240
NT = 32
EPT = E // NT
K = 128
NCH = 80
EPT_PAD = NCH * K
SB = 8
NB = NCH // SB
NEG = -1e30


def _node_stage(h_ref, wfc_ref, wl_ref, wr_ref, z_ref, asrc_ref, adst_ref):
    z = lax.dot_general(h_ref[...], wfc_ref[...], (((1,), (1,)), ((), ())),
                        preferred_element_type=jnp.float32)
    z = jnp.where(z > 0, z, 0.01 * z)
    z_ref[...] = z
    asrc_ref[...] = jnp.dot(z, wl_ref[...], preferred_element_type=jnp.float32)
    adst_ref[...] = jnp.dot(z, wr_ref[...], preferred_element_type=jnp.float32)


def _edge_score(asrc_h, adst_h, src_h, dst_h, ea_h, we_h, wm_h,
                ne_out, mx_out,
                asrc_v, adst_v, src_v, dst_v, ea_v, ne_v, we_v, wm_v, mx_v):
    cid = lax.axis_index("c")
    sid = lax.axis_index("s")
    wid = sid * 2 + cid

    pltpu.sync_copy(asrc_h, asrc_v)
    pltpu.sync_copy(adst_h, adst_v)
    pltpu.sync_copy(src_h.at[wid], src_v)
    pltpu.sync_copy(dst_h.at[wid], dst_v)
    pltpu.sync_copy(ea_h.at[wid], ea_v)
    pltpu.sync_copy(we_h, we_v)
    pltpu.sync_copy(wm_h, wm_v)

    we = we_v[...]
    wm = wm_v[...]

    def body(r, mx):
        for u in range(8):
            sl = pl.ds(u * 16, 16)
            s16 = src_v[r, sl]
            d16 = dst_v[r, sl]
            ea16 = ea_v[r, sl]
            a16 = plsc.load_gather(asrc_v, [s16]) + plsc.load_gather(adst_v, [d16])
            x = a16 * (ea16 * we)
            x = jnp.where(x > 0, x, 0.01 * x)
            ne = x * wm
            ne_v[r, sl] = ne
            mx = jnp.maximum(mx, ne)
        return mx

    mx = lax.fori_loop(0, NCH, body, jnp.full((16,), NEG, jnp.float32))
    mx_v[0, :] = mx
    pltpu.sync_copy(ne_v, ne_out.at[wid])
    pltpu.sync_copy(mx_v, mx_out.at[wid])


def _scatter_stage(z_h, src_h, dst_h, ne_h, m_h,
                   u_out, d_out,
                   acc,
                   sbuf, dbuf, nbuf, rows0, rows1, dloc, ev_s, m_v,
                   kbuf, pbuf,
                   gsem0, gsem1, ssem0, ssem1):
    cid = lax.axis_index("c")
    sid = lax.axis_index("s")
    wid = sid * 2 + cid
    rows = (rows0, rows1)
    gsem = (gsem0, gsem1)
    ssem = (ssem0, ssem1)

    pltpu.sync_copy(m_h, m_v)
    m16 = m_v[...]
    z16 = jnp.zeros((16,), jnp.float32)

    @pl.loop(0, K)
    def _zr(r):
        for u in range(8):
            rows0[r, pl.ds(u * 16, 16)] = z16

    @pl.loop(0, NCH)
    def _zd(r):
        for u in range(8):
            dloc[r, pl.ds(u * 16, 16)] = z16

    base = sid * 624
    for q in range(4):
        pltpu.sync_copy(rows0, acc.at[pl.ds(base + q * 128, 128)])
    pltpu.sync_copy(rows0.at[pl.ds(0, 112)], acc.at[pl.ds(base + 512, 112)])

    @pl.when(sid == 0)
    def _():
        pltpu.sync_copy(rows0.at[pl.ds(0, 16)], acc.at[pl.ds(9984, 16)])

    plsc.subcore_barrier()

    def wait_dma(sem, buf):
        pltpu.make_async_copy(z_h.at[pl.ds(0, K)], buf, sem).wait()

    @pl.loop(0, NB)
    def _batches(bt):
        t0 = bt * SB
        pltpu.sync_copy(src_h.at[wid, pl.ds(t0, SB)], sbuf)
        pltpu.sync_copy(dst_h.at[wid, pl.ds(t0, SB)], dbuf)
        pltpu.sync_copy(ne_h.at[wid, pl.ds(t0, SB)], nbuf)

        def issue_gather(c2):
            b = c2 % 2
            for g in range(8):
                s16 = sbuf[c2, pl.ds(g * 16, 16)]
                pltpu.async_copy(z_h.at[s16], rows[b].at[pl.ds(g * 16, 16)],
                                 gsem[b])

        def compute(c2):
            b = c2 % 2
            io16 = lax.iota(jnp.int32, 16)

            def splat(ref, idx):
                return plsc.load_gather(ref, [jnp.zeros((16,), jnp.int32) + idx])

            def scalar_of(ref, idx):
                return jnp.max(splat(ref, idx), axis=0)

            def scale(g, _):
                sl = pl.ds(g * 16, 16)

                s16g = sbuf[c2, sl]
                ssk, ssp = plsc.sort_key_val(s16g, io16)
                kbuf[...] = ssk
                pbuf[...] = ssp
                sprev = plsc.load_gather(kbuf, [jnp.maximum(io16 - 1, 0)])
                sdup = jnp.where((ssk == sprev) & (io16 > 0), 1, 0)

                @pl.when(jnp.max(sdup, axis=0) > 0)
                def _():
                    def sfix(i, _):
                        l = 15 - i
                        eq = jnp.max(jnp.where(
                            splat(kbuf, l) == splat(kbuf, l - 1), 1, 0), axis=0)

                        @pl.when(eq > 0)
                        def _():
                            jv = g * 16 + scalar_of(pbuf, l)
                            js = g * 16 + scalar_of(pbuf, l - 1)
                            for u in range(8):
                                su = pl.ds(u * 16, 16)
                                rows[b][js, su] = rows[b][jv, su]
                        return 0

                    lax.fori_loop(0, 15, sfix, 0)

                ne16 = nbuf[c2, sl]
                ev16 = jnp.exp(ne16 - m16)
                ev_s[...] = ev16

                def scale_one(l, _):
                    s16 = splat(ev_s, l)
                    j = g * 16 + l
                    for u in range(8):
                        su = pl.ds(u * 16, 16)
                        rows[b][j, su] = rows[b][j, su] * s16
                    return 0

                lax.fori_loop(0, 16, scale_one, 0)

                d16 = dbuf[c2, sl]
                sk, sp = plsc.sort_key_val(d16, io16)
                kbuf[...] = sk
                pbuf[...] = sp
                prev = plsc.load_gather(kbuf, [jnp.maximum(io16 - 1, 0)])
                dup = jnp.where((sk == prev) & (io16 > 0), 1, 0)
                has_dup = jnp.max(dup, axis=0)

                @pl.when(has_dup > 0)
                def _():
                    def dfix(l, _):
                        eq = jnp.max(jnp.where(
                            splat(kbuf, l) == splat(kbuf, l - 1), 1, 0), axis=0)

                        @pl.when(eq > 0)
                        def _():
                            pj = scalar_of(pbuf, l)
                            pm = scalar_of(pbuf, l - 1)
                            jj = g * 16 + pj
                            jm = g * 16 + pm
                            for u in range(8):
                                su = pl.ds(u * 16, 16)
                                rows[b][jj, su] = (rows[b][jj, su]
                                                   + rows[b][jm, su])
                            evv = ev_s[...]
                            evpm = splat(ev_s, pm)
                            evv = jnp.where(io16 == pj, evv + evpm, evv)
                            evv = jnp.where(io16 == pm, 0.0, evv)
                            ev_s[...] = evv
                            dd = dbuf[c2, sl]
                            dbuf[c2, sl] = jnp.where(io16 == pm, N, dd)
                        return 0

                    lax.fori_loop(1, 16, dfix, 0)

                d16b = dbuf[c2, sl]
                plsc.addupdate_scatter(
                    dloc,
                    [lax.shift_right_logical(d16b, 7),
                     lax.bitwise_and(d16b, 127)],
                    ev_s[...])
                return 0

            lax.fori_loop(0, 8, scale, 0)

        def issue_scatter(c2):
            b = c2 % 2
            for g in range(8):
                d16 = dbuf[c2, pl.ds(g * 16, 16)]
                pltpu.async_copy(rows[b].at[pl.ds(g * 16, 16)], acc.at[d16],
                                 ssem[b], add=True)

        issue_gather(0)
        for c2 in range(SB):
            b = c2 % 2
            nb2 = 1 - b
            wait_dma(gsem[b], rows[b])
            if c2 + 1 < SB:
                if c2 >= 1:
                    wait_dma(ssem[nb2], rows[nb2])
                issue_gather(c2 + 1)
            compute(c2)
            issue_scatter(c2)
        wait_dma(ssem[0], rows0)
        wait_dma(ssem[1], rows1)

    plsc.subcore_barrier()

    for q in range(4):
        pltpu.sync_copy(acc.at[pl.ds(base + q * 128, 128)],
                        u_out.at[cid, pl.ds(base + q * 128, 128)])
    pltpu.sync_copy(acc.at[pl.ds(base + 512, 112)],
                    u_out.at[cid, pl.ds(base + 512, 112)])

    @pl.when(sid == 0)
    def _():
        pltpu.sync_copy(acc.at[pl.ds(9984, 16)],
                        u_out.at[cid, pl.ds(9984, 16)])

    pltpu.sync_copy(dloc, d_out.at[wid])


def _combine(u_ref, d_ref, out_ref):
    s = u_ref[0] + u_ref[1]
    dd = jnp.sum(d_ref[0], axis=0)
    safe = jnp.where(dd > 0, dd, 1.0)
    out_ref[...] = s / safe[:, None]


@jax.jit
def kernel(h, edge_index, edge_attr, W_fc, W_attn, W_edge, W_m):
    hp = jnp.zeros((NPAD, D), jnp.float32).at[:N].set(h)
    wl = W_attn[0, :D].reshape(D, 1)
    wr = W_attn[0, D:].reshape(D, 1)

    BR = 1024
    z, asrc, adst = pl.pallas_call(
        _node_stage,
        grid=(NPAD // BR,),
        in_specs=[
            pl.BlockSpec((BR, D), lambda i: (i, 0)),
            pl.BlockSpec((D, D), lambda i: (0, 0)),
            pl.BlockSpec((D, 1), lambda i: (0, 0)),
            pl.BlockSpec((D, 1), lambda i: (0, 0)),
        ],
        out_specs=[
            pl.BlockSpec((BR, D), lambda i: (i, 0)),
            pl.BlockSpec((BR, 1), lambda i: (i, 0)),
            pl.BlockSpec((BR, 1), lambda i: (i, 0)),
        ],
        out_shape=[
            jax.ShapeDtypeStruct((NPAD, D), jnp.float32),
            jax.ShapeDtypeStruct((NPAD, 1), jnp.float32),
            jax.ShapeDtypeStruct((NPAD, 1), jnp.float32),
        ],
    )(hp, W_fc, wl, wr)

    pad_i = jnp.zeros((NT, EPT_PAD - EPT), jnp.int32)
    pad_f = jnp.zeros((NT, EPT_PAD - EPT), jnp.float32)
    src_p = jnp.concatenate(
        [edge_index[0].reshape(NT, EPT), pad_i], axis=1).reshape(NT, NCH, K)
    dst_p = jnp.concatenate(
        [edge_index[1].reshape(NT, EPT), pad_i], axis=1).reshape(NT, NCH, K)
    ea_p = jnp.concatenate(
        [edge_attr[:, 0].reshape(NT, EPT), pad_f], axis=1).reshape(NT, NCH, K)
    we16 = jnp.full((16,), W_edge[0, 0], jnp.float32)
    wm16 = jnp.full((16,), W_m[0, 0], jnp.float32)

    mesh = plsc.VectorSubcoreMesh(core_axis_name="c", subcore_axis_name="s")
    sc_params = pltpu.CompilerParams(needs_layout_passes=False, use_tc_tiling_on_sc=False)
    ne_r, mx_r = pl.kernel(
        _edge_score,
        out_type=[
            jax.ShapeDtypeStruct((NT, NCH, K), jnp.float32),
            jax.ShapeDtypeStruct((NT, 1, 16), jnp.float32),
        ],
        mesh=mesh,
        scratch_types=[
            pltpu.VMEM((NPAD,), jnp.float32),
            pltpu.VMEM((NPAD,), jnp.float32),
            pltpu.VMEM((NCH, K), jnp.int32),
            pltpu.VMEM((NCH, K), jnp.int32),
            pltpu.VMEM((NCH, K), jnp.float32),
            pltpu.VMEM((NCH, K), jnp.float32),
            pltpu.VMEM((16,), jnp.float32),
            pltpu.VMEM((16,), jnp.float32),
            pltpu.VMEM((1, 16), jnp.float32),
        ],
        compiler_params=sc_params,
    )(asrc[:, 0], adst[:, 0], src_p, dst_p, ea_p, we16, wm16)

    M = jnp.max(mx_r)
    m16 = jnp.full((16,), M, jnp.float32)
    ne_p = ne_r.reshape(NT, EPT_PAD).at[:, EPT:].set(NEG).reshape(NT, NCH, K)

    u_par, d_par = pl.kernel(
        _scatter_stage,
        out_type=[
            jax.ShapeDtypeStruct((2, N, D), jnp.float32),
            jax.ShapeDtypeStruct((NT, NCH, K), jnp.float32),
        ],
        mesh=mesh,
        scratch_types=[
            pltpu.VMEM_SHARED((N + 8, D), jnp.float32),
            pltpu.VMEM((SB, K), jnp.int32),
            pltpu.VMEM((SB, K), jnp.int32),
            pltpu.VMEM((SB, K), jnp.float32),
            pltpu.VMEM((K, D), jnp.float32),
            pltpu.VMEM((K, D), jnp.float32),
            pltpu.VMEM((NCH, K), jnp.float32),
            pltpu.VMEM((16,), jnp.float32),
            pltpu.VMEM((16,), jnp.float32),
            pltpu.VMEM((16,), jnp.int32),
            pltpu.VMEM((16,), jnp.int32),
            pltpu.SemaphoreType.DMA,
            pltpu.SemaphoreType.DMA,
            pltpu.SemaphoreType.DMA,
            pltpu.SemaphoreType.DMA,
        ],
        compiler_params=sc_params,
    )(z, src_p, dst_p, ne_p, m16)

    BN = 1000
    d_blk = d_par.reshape(NT, EPT_PAD)[:, :N].reshape(
        NT, N // BN, BN).transpose(1, 0, 2)

    h_out = pl.pallas_call(
        _combine,
        grid=(N // BN,),
        in_specs=[
            pl.BlockSpec((2, BN, D), lambda i: (0, i, 0)),
            pl.BlockSpec((1, NT, BN), lambda i: (i, 0, 0)),
        ],
        out_specs=pl.BlockSpec((BN, D), lambda i: (i, 0)),
        out_shape=jax.ShapeDtypeStruct((N, D), jnp.float32),
    )(u_par, d_blk)
    return h_out


# speedup vs baseline: 13.8974x; 1.0001x over previous
import functools
import jax
import jax.numpy as jnp
from jax import lax
from jax.experimental import pallas as pl
from jax.experimental.pallas import tpu as pltpu
from jax.experimental.pallas import tpu_sc as plsc

N = 10000
E = 320000
D = 128
NPAD = 10240
NT = 32
EPT = E // NT
K = 128
NCH = 80
EPT_PAD = NCH * K
SB = 8
NB = NCH // SB
NEG = -1e30


def _node_stage(h_ref, wfc_ref, wl_ref, wr_ref, z_ref, asrc_ref, adst_ref):
    z = lax.dot_general(h_ref[...], wfc_ref[...], (((1,), (1,)), ((), ())),
                        preferred_element_type=jnp.float32)
    z = jnp.where(z > 0, z, 0.01 * z)
    z_ref[...] = z
    asrc_ref[...] = jnp.dot(z, wl_ref[...], preferred_element_type=jnp.float32)
    adst_ref[...] = jnp.dot(z, wr_ref[...], preferred_element_type=jnp.float32)


def _edge_score(asrc_h, adst_h, src_h, dst_h, ea_h, we_h, wm_h,
                ne_out, mx_out,
                asrc_v, adst_v, src_v, dst_v, ea_v, ne_v, we_v, wm_v, mx_v):
    cid = lax.axis_index("c")
    sid = lax.axis_index("s")
    wid = sid * 2 + cid

    pltpu.sync_copy(asrc_h, asrc_v)
    pltpu.sync_copy(adst_h, adst_v)
    pltpu.sync_copy(src_h.at[wid], src_v)
    pltpu.sync_copy(dst_h.at[wid], dst_v)
    pltpu.sync_copy(ea_h.at[wid], ea_v)
    pltpu.sync_copy(we_h, we_v)
    pltpu.sync_copy(wm_h, wm_v)

    we = we_v[...]
    wm = wm_v[...]

    def body(r, mx):
        for u in range(8):
            sl = pl.ds(u * 16, 16)
            s16 = src_v[r, sl]
            d16 = dst_v[r, sl]
            ea16 = ea_v[r, sl]
            a16 = plsc.load_gather(asrc_v, [s16]) + plsc.load_gather(adst_v, [d16])
            x = a16 * (ea16 * we)
            x = jnp.where(x > 0, x, 0.01 * x)
            ne = x * wm
            ne_v[r, sl] = ne
            mx = jnp.maximum(mx, ne)
        return mx

    mx = lax.fori_loop(0, NCH, body, jnp.full((16,), NEG, jnp.float32))
    mx_v[0, :] = mx
    pltpu.sync_copy(ne_v, ne_out.at[wid])
    pltpu.sync_copy(mx_v, mx_out.at[wid])


def _scatter_stage(z_h, src_h, dst_h, ne_h, m_h,
                   u_out, d_out,
                   acc,
                   sbuf, dbuf, nbuf, rows0, rows1, dloc, ev_s, m_v,
                   kbuf, pbuf,
                   gsem0, gsem1, ssem0, ssem1):
    cid = lax.axis_index("c")
    sid = lax.axis_index("s")
    wid = sid * 2 + cid
    rows = (rows0, rows1)
    gsem = (gsem0, gsem1)
    ssem = (ssem0, ssem1)

    pltpu.sync_copy(m_h, m_v)
    m16 = m_v[...]
    z16 = jnp.zeros((16,), jnp.float32)

    @pl.loop(0, K)
    def _zr(r):
        for u in range(8):
            rows0[r, pl.ds(u * 16, 16)] = z16

    @pl.loop(0, NCH)
    def _zd(r):
        for u in range(8):
            dloc[r, pl.ds(u * 16, 16)] = z16

    base = sid * 624
    for q in range(4):
        pltpu.sync_copy(rows0, acc.at[pl.ds(base + q * 128, 128)])
    pltpu.sync_copy(rows0.at[pl.ds(0, 112)], acc.at[pl.ds(base + 512, 112)])

    @pl.when(sid == 0)
    def _():
        pltpu.sync_copy(rows0.at[pl.ds(0, 16)], acc.at[pl.ds(9984, 16)])

    plsc.subcore_barrier()

    def wait_dma(sem, buf):
        pltpu.make_async_copy(z_h.at[pl.ds(0, K)], buf, sem).wait()

    @pl.loop(0, NB)
    def _batches(bt):
        t0 = bt * SB
        pltpu.sync_copy(src_h.at[wid, pl.ds(t0, SB)], sbuf)
        pltpu.sync_copy(dst_h.at[wid, pl.ds(t0, SB)], dbuf)
        pltpu.sync_copy(ne_h.at[wid, pl.ds(t0, SB)], nbuf)

        def issue_gather(c2):
            b = c2 % 2
            for g in range(8):
                s16 = sbuf[c2, pl.ds(g * 16, 16)]
                pltpu.async_copy(z_h.at[s16], rows[b].at[pl.ds(g * 16, 16)],
                                 gsem[b])

        def compute(c2):
            b = c2 % 2
            io16 = lax.iota(jnp.int32, 16)

            def splat(ref, idx):
                return plsc.load_gather(ref, [jnp.zeros((16,), jnp.int32) + idx])

            def scalar_of(ref, idx):
                return jnp.max(splat(ref, idx), axis=0)

            def scale(g, _):
                sl = pl.ds(g * 16, 16)

                s16g = sbuf[c2, sl]
                ssk, ssp = plsc.sort_key_val(s16g, io16)
                kbuf[...] = ssk
                pbuf[...] = ssp
                sprev = plsc.load_gather(kbuf, [jnp.maximum(io16 - 1, 0)])
                sdup = jnp.where((ssk == sprev) & (io16 > 0), 1, 0)

                @pl.when(jnp.max(sdup, axis=0) > 0)
                def _():
                    def sfix(i, _):
                        l = 15 - i
                        eq = jnp.max(jnp.where(
                            splat(kbuf, l) == splat(kbuf, l - 1), 1, 0), axis=0)

                        @pl.when(eq > 0)
                        def _():
                            jv = g * 16 + scalar_of(pbuf, l)
                            js = g * 16 + scalar_of(pbuf, l - 1)
                            for u in range(8):
                                su = pl.ds(u * 16, 16)
                                rows[b][js, su] = rows[b][jv, su]
                        return 0

                    lax.fori_loop(0, 15, sfix, 0)

                ne16 = nbuf[c2, sl]
                ev16 = jnp.exp(ne16 - m16)
                ev_s[...] = ev16

                def scale_one(l, _):
                    s16 = splat(ev_s, l)
                    j = g * 16 + l
                    for u in range(8):
                        su = pl.ds(u * 16, 16)
                        rows[b][j, su] = rows[b][j, su] * s16
                    return 0

                lax.fori_loop(0, 16, scale_one, 0, unroll=4)

                d16 = dbuf[c2, sl]
                sk, sp = plsc.sort_key_val(d16, io16)
                kbuf[...] = sk
                pbuf[...] = sp
                prev = plsc.load_gather(kbuf, [jnp.maximum(io16 - 1, 0)])
                dup = jnp.where((sk == prev) & (io16 > 0), 1, 0)
                has_dup = jnp.max(dup, axis=0)

                @pl.when(has_dup > 0)
                def _():
                    def dfix(l, _):
                        eq = jnp.max(jnp.where(
                            splat(kbuf, l) == splat(kbuf, l - 1), 1, 0), axis=0)

                        @pl.when(eq > 0)
                        def _():
                            pj = scalar_of(pbuf, l)
                            pm = scalar_of(pbuf, l - 1)
                            jj = g * 16 + pj
                            jm = g * 16 + pm
                            for u in range(8):
                                su = pl.ds(u * 16, 16)
                                rows[b][jj, su] = (rows[b][jj, su]
                                                   + rows[b][jm, su])
                            evv = ev_s[...]
                            evpm = splat(ev_s, pm)
                            evv = jnp.where(io16 == pj, evv + evpm, evv)
                            evv = jnp.where(io16 == pm, 0.0, evv)
                            ev_s[...] = evv
                            dd = dbuf[c2, sl]
                            dbuf[c2, sl] = jnp.where(io16 == pm, N, dd)
                        return 0

                    lax.fori_loop(1, 16, dfix, 0)

                d16b = dbuf[c2, sl]
                plsc.addupdate_scatter(
                    dloc,
                    [lax.shift_right_logical(d16b, 7),
                     lax.bitwise_and(d16b, 127)],
                    ev_s[...])
                return 0

            lax.fori_loop(0, 8, scale, 0)

        def issue_scatter(c2):
            b = c2 % 2
            for g in range(8):
                d16 = dbuf[c2, pl.ds(g * 16, 16)]
                pltpu.async_copy(rows[b].at[pl.ds(g * 16, 16)], acc.at[d16],
                                 ssem[b], add=True)

        issue_gather(0)
        for c2 in range(SB):
            b = c2 % 2
            nb2 = 1 - b
            wait_dma(gsem[b], rows[b])
            if c2 + 1 < SB:
                if c2 >= 1:
                    wait_dma(ssem[nb2], rows[nb2])
                issue_gather(c2 + 1)
            compute(c2)
            issue_scatter(c2)
        wait_dma(ssem[0], rows0)
        wait_dma(ssem[1], rows1)

    plsc.subcore_barrier()

    for q in range(4):
        pltpu.sync_copy(acc.at[pl.ds(base + q * 128, 128)],
                        u_out.at[cid, pl.ds(base + q * 128, 128)])
    pltpu.sync_copy(acc.at[pl.ds(base + 512, 112)],
                    u_out.at[cid, pl.ds(base + 512, 112)])

    @pl.when(sid == 0)
    def _():
        pltpu.sync_copy(acc.at[pl.ds(9984, 16)],
                        u_out.at[cid, pl.ds(9984, 16)])

    pltpu.sync_copy(dloc, d_out.at[wid])


def _combine(u_ref, d_ref, out_ref):
    s = u_ref[0] + u_ref[1]
    dd = jnp.sum(d_ref[0], axis=0)
    safe = jnp.where(dd > 0, dd, 1.0)
    out_ref[...] = s / safe[:, None]


@jax.jit
def kernel(h, edge_index, edge_attr, W_fc, W_attn, W_edge, W_m):
    hp = jnp.zeros((NPAD, D), jnp.float32).at[:N].set(h)
    wl = W_attn[0, :D].reshape(D, 1)
    wr = W_attn[0, D:].reshape(D, 1)

    BR = 1024
    z, asrc, adst = pl.pallas_call(
        _node_stage,
        grid=(NPAD // BR,),
        in_specs=[
            pl.BlockSpec((BR, D), lambda i: (i, 0)),
            pl.BlockSpec((D, D), lambda i: (0, 0)),
            pl.BlockSpec((D, 1), lambda i: (0, 0)),
            pl.BlockSpec((D, 1), lambda i: (0, 0)),
        ],
        out_specs=[
            pl.BlockSpec((BR, D), lambda i: (i, 0)),
            pl.BlockSpec((BR, 1), lambda i: (i, 0)),
            pl.BlockSpec((BR, 1), lambda i: (i, 0)),
        ],
        out_shape=[
            jax.ShapeDtypeStruct((NPAD, D), jnp.float32),
            jax.ShapeDtypeStruct((NPAD, 1), jnp.float32),
            jax.ShapeDtypeStruct((NPAD, 1), jnp.float32),
        ],
    )(hp, W_fc, wl, wr)

    pad_i = jnp.zeros((NT, EPT_PAD - EPT), jnp.int32)
    pad_f = jnp.zeros((NT, EPT_PAD - EPT), jnp.float32)
    src_p = jnp.concatenate(
        [edge_index[0].reshape(NT, EPT), pad_i], axis=1).reshape(NT, NCH, K)
    dst_p = jnp.concatenate(
        [edge_index[1].reshape(NT, EPT), pad_i], axis=1).reshape(NT, NCH, K)
    ea_p = jnp.concatenate(
        [edge_attr[:, 0].reshape(NT, EPT), pad_f], axis=1).reshape(NT, NCH, K)
    we16 = jnp.full((16,), W_edge[0, 0], jnp.float32)
    wm16 = jnp.full((16,), W_m[0, 0], jnp.float32)

    mesh = plsc.VectorSubcoreMesh(core_axis_name="c", subcore_axis_name="s")
    sc_params = pltpu.CompilerParams(needs_layout_passes=False, use_tc_tiling_on_sc=False)
    ne_r, mx_r = pl.kernel(
        _edge_score,
        out_type=[
            jax.ShapeDtypeStruct((NT, NCH, K), jnp.float32),
            jax.ShapeDtypeStruct((NT, 1, 16), jnp.float32),
        ],
        mesh=mesh,
        scratch_types=[
            pltpu.VMEM((NPAD,), jnp.float32),
            pltpu.VMEM((NPAD,), jnp.float32),
            pltpu.VMEM((NCH, K), jnp.int32),
            pltpu.VMEM((NCH, K), jnp.int32),
            pltpu.VMEM((NCH, K), jnp.float32),
            pltpu.VMEM((NCH, K), jnp.float32),
            pltpu.VMEM((16,), jnp.float32),
            pltpu.VMEM((16,), jnp.float32),
            pltpu.VMEM((1, 16), jnp.float32),
        ],
        compiler_params=sc_params,
    )(asrc[:, 0], adst[:, 0], src_p, dst_p, ea_p, we16, wm16)

    M = jnp.max(mx_r)
    m16 = jnp.full((16,), M, jnp.float32)
    ne_p = ne_r.reshape(NT, EPT_PAD).at[:, EPT:].set(NEG).reshape(NT, NCH, K)

    u_par, d_par = pl.kernel(
        _scatter_stage,
        out_type=[
            jax.ShapeDtypeStruct((2, N, D), jnp.float32),
            jax.ShapeDtypeStruct((NT, NCH, K), jnp.float32),
        ],
        mesh=mesh,
        scratch_types=[
            pltpu.VMEM_SHARED((N + 8, D), jnp.float32),
            pltpu.VMEM((SB, K), jnp.int32),
            pltpu.VMEM((SB, K), jnp.int32),
            pltpu.VMEM((SB, K), jnp.float32),
            pltpu.VMEM((K, D), jnp.float32),
            pltpu.VMEM((K, D), jnp.float32),
            pltpu.VMEM((NCH, K), jnp.float32),
            pltpu.VMEM((16,), jnp.float32),
            pltpu.VMEM((16,), jnp.float32),
            pltpu.VMEM((16,), jnp.int32),
            pltpu.VMEM((16,), jnp.int32),
            pltpu.SemaphoreType.DMA,
            pltpu.SemaphoreType.DMA,
            pltpu.SemaphoreType.DMA,
            pltpu.SemaphoreType.DMA,
        ],
        compiler_params=sc_params,
    )(z, src_p, dst_p, ne_p, m16)

    BN = 1000
    d_blk = d_par.reshape(NT, EPT_PAD)[:, :N].reshape(
        NT, N // BN, BN).transpose(1, 0, 2)

    h_out = pl.pallas_call(
        _combine,
        grid=(N // BN,),
        in_specs=[
            pl.BlockSpec((2, BN, D), lambda i: (0, i, 0)),
            pl.BlockSpec((1, NT, BN), lambda i: (i, 0, 0)),
        ],
        out_specs=pl.BlockSpec((BN, D), lambda i: (i, 0)),
        out_shape=jax.ShapeDtypeStruct((N, D), jnp.float32),
    )(u_par, d_blk)
    return h_out
